# baseline (speedup 1.0000x reference)
import jax
import jax.numpy as jnp
from jax import lax
from jax.experimental import pallas as pl
from jax.experimental.pallas import tpu as pltpu
from jax.experimental.pallas import tpu_sc as plsc

B = 16384
R_IN = 26
R_OUT = 11
D = 64
IDX_LIST = (0, 1, 6, 12, 13, 14, 15, 17, 20, 21, 22)

NC, NS = 2, 16
NW = NC * NS
ROWS = B * R_OUT
RPW = ROWS // NW
G = 128
NG = RPW // G
NSLOT = 2
AHEAD = NSLOT - 1


def _sc_body(tab_hbm, rid_hbm, out_hbm, idxbuf, rowbuf, *sems):
    rsems, wsems = sems[:NSLOT], sems[NSLOT:]
    wid = lax.axis_index("s") * NC + lax.axis_index("c")
    base = wid * RPW
    pltpu.sync_copy(rid_hbm.at[wid], idxbuf)

    def gather(c):
        s = c % NSLOT
        return pltpu.async_copy(
            tab_hbm.at[idxbuf.at[c]], rowbuf.at[s], rsems[s]
        )

    reads = [None] * NSLOT
    writes = [None] * NSLOT
    for c in range(NG + AHEAD):
        if c < NG:
            s = c % NSLOT
            if writes[s] is not None:
                writes[s].wait()
            reads[s] = gather(c)
        d = c - AHEAD
        if 0 <= d < NG:
            s = d % NSLOT
            reads[s].wait()
            writes[s] = pltpu.async_copy(
                rowbuf.at[s], out_hbm.at[pl.ds(base + d * G, G)], wsems[s]
            )
    for h in writes:
        if h is not None:
            h.wait()


@jax.jit
def kernel(inputs):
    tab = inputs.reshape(B * R_IN, D)
    rid = (
        R_IN * jnp.arange(B, dtype=jnp.int32)[:, None]
        + jnp.array(IDX_LIST, dtype=jnp.int32)[None, :]
    ).reshape(NW, NG, G)
    mesh = plsc.VectorSubcoreMesh(core_axis_name="c", subcore_axis_name="s")
    out2 = pl.kernel(
        _sc_body,
        out_type=jax.ShapeDtypeStruct((ROWS, D), jnp.float32),
        mesh=mesh,
        scratch_types=[
            pltpu.VMEM((NG, G), jnp.int32),
            pltpu.VMEM((NSLOT, G, D), jnp.float32),
        ]
        + [pltpu.SemaphoreType.DMA] * (2 * NSLOT),
        compiler_params=pltpu.CompilerParams(use_tc_tiling_on_sc=False),
    )(tab, rid)
    return out2.reshape(B, R_OUT, D)


# speedup vs baseline: 1.1651x; 1.1651x over previous
import jax
import jax.numpy as jnp
from jax import lax
from jax.experimental import pallas as pl
from jax.experimental.pallas import tpu as pltpu
from jax.experimental.pallas import tpu_sc as plsc

B = 16384
R_IN = 26
R_OUT = 11
D = 64
RUNS = ((0, 2, 0), (6, 1, 2), (12, 4, 3), (17, 1, 7), (20, 3, 8))

NC, NS = 2, 16
NW = NC * NS
BPW = B // NW
NB = 16
NCHUNK = BPW // NB

W_IN = R_IN * D
W_OUT = R_OUT * D


def _sc_body(in_hbm, out_hbm, ibuf, obuf):
    wid = lax.axis_index("s") * NC + lax.axis_index("c")
    base = wid * BPW

    def compact_row(i, carry):
        for (src, w, dst) in RUNS:
            obuf[i, pl.ds(dst * D, w * D)] = ibuf[i, pl.ds(src * D, w * D)]
        return carry

    def chunk_body(c, carry):
        b0 = base + c * NB
        pltpu.sync_copy(in_hbm.at[pl.ds(b0, NB)], ibuf)
        lax.fori_loop(0, NB, compact_row, 0)
        pltpu.sync_copy(obuf, out_hbm.at[pl.ds(b0, NB)])
        return carry

    lax.fori_loop(0, NCHUNK, chunk_body, 0)


@jax.jit
def kernel(inputs):
    in2 = inputs.reshape(B, W_IN)
    mesh = plsc.VectorSubcoreMesh(core_axis_name="c", subcore_axis_name="s")
    out2 = pl.kernel(
        _sc_body,
        out_type=jax.ShapeDtypeStruct((B, W_OUT), jnp.float32),
        mesh=mesh,
        scratch_types=[
            pltpu.VMEM((NB, W_IN), jnp.float32),
            pltpu.VMEM((NB, W_OUT), jnp.float32),
        ],
        compiler_params=pltpu.CompilerParams(use_tc_tiling_on_sc=False),
    )(in2)
    return out2.reshape(B, R_OUT, D)


# speedup vs baseline: 1.3196x; 1.1326x over previous
import jax
import jax.numpy as jnp
from jax import lax
from jax.experimental import pallas as pl
from jax.experimental.pallas import tpu as pltpu
from jax.experimental.pallas import tpu_sc as plsc

B = 16384
R_IN = 26
R_OUT = 11
R_HI = 23
D = 64
RUNS = ((0, 2, 0), (6, 1, 2), (12, 4, 3), (17, 1, 7), (20, 3, 8))

NC, NS = 2, 16
NW = NC * NS
BPW = B // NW
NB = 16
NCHUNK = BPW // NB
NPAIR = NCHUNK // 2

W_IN = R_IN * D
W_HI = R_HI * D
W_OUT = R_OUT * D


def _sc_body(in_hbm, out_hbm, ibuf, obuf, rs0, rs1, ws0, ws1):
    rsems = (rs0, rs1)
    wsems = (ws0, ws1)
    wid = lax.axis_index("s") * NC + lax.axis_index("c")
    base = wid * BPW

    def read(c, k):
        pltpu.async_copy(
            in_hbm.at[pl.ds(base + c * NB, NB), pl.ds(0, W_HI)],
            ibuf.at[k],
            rsems[k],
        )

    def wait_read(k):
        pltpu.make_async_copy(
            in_hbm.at[pl.ds(0, NB), pl.ds(0, W_HI)], ibuf.at[k], rsems[k]
        ).wait()

    def write(c, k):
        pltpu.async_copy(obuf.at[k], out_hbm.at[pl.ds(base + c * NB, NB)], wsems[k])

    def wait_write(k):
        pltpu.make_async_copy(
            obuf.at[k], out_hbm.at[pl.ds(0, NB)], wsems[k]
        ).wait()

    def compact(k):
        @plsc.parallel_loop(0, NB, unroll=2)
        def _(i):
            for (src, w, dst) in RUNS:
                obuf[k, i, pl.ds(dst * D, w * D)] = ibuf[k, i, pl.ds(src * D, w * D)]

    read(0, 0)
    read(1, 1)
    for k in (0, 1):
        wait_read(k)
        compact(k)
        write(k, k)
        read(2 + k, k)

    def pair_body(g, carry):
        for k in (0, 1):
            c = 2 * g + k
            wait_read(k)
            wait_write(k)
            compact(k)
            write(c, k)
            read(c + 2, k)
        return carry

    lax.fori_loop(1, NPAIR - 1, pair_body, 0)

    for k in (0, 1):
        c = NCHUNK - 2 + k
        wait_read(k)
        wait_write(k)
        compact(k)
        write(c, k)
    for k in (0, 1):
        wait_write(k)


@jax.jit
def kernel(inputs):
    in2 = inputs.reshape(B, W_IN)
    mesh = plsc.VectorSubcoreMesh(core_axis_name="c", subcore_axis_name="s")
    out2 = pl.kernel(
        _sc_body,
        out_type=jax.ShapeDtypeStruct((B, W_OUT), jnp.float32),
        mesh=mesh,
        scratch_types=[
            pltpu.VMEM((2, NB, W_HI), jnp.float32),
            pltpu.VMEM((2, NB, W_OUT), jnp.float32),
            pltpu.SemaphoreType.DMA,
            pltpu.SemaphoreType.DMA,
            pltpu.SemaphoreType.DMA,
            pltpu.SemaphoreType.DMA,
        ],
        compiler_params=pltpu.CompilerParams(use_tc_tiling_on_sc=False),
    )(in2)
    return out2.reshape(B, R_OUT, D)


# speedup vs baseline: 2.1336x; 1.6168x over previous
import jax
import jax.numpy as jnp
from jax.experimental import pallas as pl
from jax.experimental.pallas import tpu as pltpu

B = 16384
R_IN = 26
R_OUT = 11
D = 64
RUNS = ((0, 2, 0), (6, 1, 2), (12, 4, 3), (17, 1, 7), (20, 3, 8))

W_IN = R_IN * D
W_OUT = R_OUT * D
BN = 256


def _tc_body(in_ref, out_ref):
    for (src, w, dst) in RUNS:
        out_ref[:, pl.ds(dst * D, w * D)] = in_ref[:, pl.ds(src * D, w * D)]


@jax.jit
def kernel(inputs):
    in2 = inputs.reshape(B, W_IN)
    out2 = pl.pallas_call(
        _tc_body,
        grid=(B // BN,),
        in_specs=[pl.BlockSpec((BN, W_IN), lambda i: (i, 0))],
        out_specs=pl.BlockSpec((BN, W_OUT), lambda i: (i, 0)),
        out_shape=jax.ShapeDtypeStruct((B, W_OUT), jnp.float32),
        compiler_params=pltpu.CompilerParams(
            dimension_semantics=("arbitrary",),
        ),
    )(in2)
    return out2.reshape(B, R_OUT, D)


# speedup vs baseline: 2.2119x; 1.0367x over previous
import jax
import jax.numpy as jnp
from jax.experimental import pallas as pl
from jax.experimental.pallas import tpu as pltpu

B = 16384
R_IN = 26
R_OUT = 11
D = 64
PIECES = (
    (0, 2, 0, 2, 0),
    (6, 2, 0, 1, 2),
    (12, 4, 0, 4, 3),
    (16, 2, 1, 1, 7),
    (20, 2, 0, 2, 8),
    (22, 2, 0, 1, 10),
)

W_IN = R_IN * D
W_OUT = R_OUT * D
BN = 256


def _tc_body(*refs):
    ins, out_ref = refs[:-1], refs[-1]
    for r, (_, _, off, take, dst) in zip(ins, PIECES):
        out_ref[:, pl.ds(dst * D, take * D)] = r[:, pl.ds(off * D, take * D)]


def _spec(src, w):
    return pl.BlockSpec((BN, w * D), lambda i, s=src // w: (i, s))


@jax.jit
def kernel(inputs):
    in2 = inputs.reshape(B, W_IN)
    out2 = pl.pallas_call(
        _tc_body,
        grid=(B // BN,),
        in_specs=[_spec(src, w) for (src, w, _, _, _) in PIECES],
        out_specs=pl.BlockSpec((BN, W_OUT), lambda i: (i, 0)),
        out_shape=jax.ShapeDtypeStruct((B, W_OUT), jnp.float32),
        compiler_params=pltpu.CompilerParams(
            dimension_semantics=("arbitrary",),
        ),
    )(*([in2] * len(PIECES)))
    return out2.reshape(B, R_OUT, D)


# speedup vs baseline: 2.5176x; 1.1382x over previous
import jax
import jax.numpy as jnp
from jax.experimental import pallas as pl
from jax.experimental.pallas import tpu as pltpu

B = 16384
R_IN = 26
R_OUT = 11
D = 64
PIECES = (
    (0, 2, 0, 2, 0),
    (6, 2, 0, 1, 2),
    (12, 4, 0, 4, 3),
    (16, 2, 1, 1, 7),
    (20, 2, 0, 2, 8),
    (22, 2, 0, 1, 10),
)

W_IN = R_IN * D
W_OUT = R_OUT * D
BN = 1024


def _tc_body(*refs):
    ins, out_ref = refs[:-1], refs[-1]
    for r, (_, _, off, take, dst) in zip(ins, PIECES):
        out_ref[:, pl.ds(dst * D, take * D)] = r[:, pl.ds(off * D, take * D)]


def _spec(src, w):
    return pl.BlockSpec((BN, w * D), lambda i, s=src // w: (i, s))


@jax.jit
def kernel(inputs):
    in2 = inputs.reshape(B, W_IN)
    out2 = pl.pallas_call(
        _tc_body,
        grid=(B // BN,),
        in_specs=[_spec(src, w) for (src, w, _, _, _) in PIECES],
        out_specs=pl.BlockSpec((BN, W_OUT), lambda i: (i, 0)),
        out_shape=jax.ShapeDtypeStruct((B, W_OUT), jnp.float32),
        compiler_params=pltpu.CompilerParams(
            dimension_semantics=("arbitrary",),
        ),
    )(*([in2] * len(PIECES)))
    return out2.reshape(B, R_OUT, D)


# speedup vs baseline: 2.5314x; 1.0055x over previous
import jax
import jax.numpy as jnp
from jax.experimental import pallas as pl
from jax.experimental.pallas import tpu as pltpu

B = 16384
R_IN = 26
R_OUT = 11
D = 64
PIECES = (
    (0, 2, ((0, 2, 0),)),
    (6, 2, ((0, 1, 2),)),
    (12, 6, ((0, 4, 3), (5, 1, 7))),
    (20, 4, ((0, 3, 8),)),
)

W_IN = R_IN * D
W_OUT = R_OUT * D
BN = 2048


def _tc_body(*refs):
    ins, out_ref = refs[:-1], refs[-1]
    for r, (_, _, takes) in zip(ins, PIECES):
        for (off, take, dst) in takes:
            out_ref[:, pl.ds(dst * D, take * D)] = r[:, pl.ds(off * D, take * D)]


def _spec(src, w):
    return pl.BlockSpec((BN, w * D), lambda i, s=src // w: (i, s))


@jax.jit
def kernel(inputs):
    in2 = inputs.reshape(B, W_IN)
    out2 = pl.pallas_call(
        _tc_body,
        grid=(B // BN,),
        in_specs=[_spec(src, w) for (src, w, _) in PIECES],
        out_specs=pl.BlockSpec((BN, W_OUT), lambda i: (i, 0)),
        out_shape=jax.ShapeDtypeStruct((B, W_OUT), jnp.float32),
        compiler_params=pltpu.CompilerParams(
            dimension_semantics=("arbitrary",),
        ),
    )(*([in2] * len(PIECES)))
    return out2.reshape(B, R_OUT, D)
